# Initial kernel scaffold; baseline (speedup 1.0000x reference)
import jax
import jax.numpy as jnp
from jax.experimental import pallas as pl


def kernel(x, atom_table, pos_table, graph_token):
    raise NotImplementedError("write your pallas kernel here")


# speedup vs baseline: 1.0004x; 1.0004x over previous
import jax
import jax.numpy as jnp
from jax import lax
from jax.experimental import pallas as pl
from jax.experimental.pallas import tpu as pltpu
from jax.experimental.pallas import tpu_sc as plsc

B, L, H = 16, 1024, 768
HALF = L // 2
CHUNK = 64
NCHUNK = HALF // CHUNK
LANES = 16


def _positions_body(x_ref, out_ref):
    mask = (x_ref[...] != 0)
    tri = (lax.broadcasted_iota(jnp.int32, (L, L), 0)
           <= lax.broadcasted_iota(jnp.int32, (L, L), 1)).astype(jnp.float32)
    cs = jax.lax.dot_general(mask.astype(jnp.float32), tri,
                             (((1,), (0,)), ((), ())),
                             preferred_element_type=jnp.float32)
    out_ref[...] = cs.astype(jnp.int32) * mask.astype(jnp.int32)


def _positions(x):
    return pl.pallas_call(
        _positions_body,
        out_shape=jax.ShapeDtypeStruct((B, L), jnp.int32),
    )(x)


def _sc_body(x_hbm, posn_hbm, atom_hbm, pos_hbm, gt_hbm, out_hbm,
             aidx, pidx, oidx, gidx, abuf, pbuf, gtbuf,
             sema, semp, semo):
    c = lax.axis_index("c")
    s = lax.axis_index("s")
    b = s
    half = c
    iota = lax.iota(jnp.int32, LANES)

    base = b * (L + 1) + 1 + half * HALF
    for k in range(NCHUNK):
        off = half * HALF + k * CHUNK
        pltpu.sync_copy(x_hbm.at[b, pl.ds(off, CHUNK)], aidx.at[k])
        pltpu.sync_copy(posn_hbm.at[b, pl.ds(off, CHUNK)], pidx.at[k])
        for j in range(CHUNK // LANES):
            oidx[k, pl.ds(j * LANES, LANES)] = base + k * CHUNK + j * LANES + iota

    for k in range(NCHUNK):
        ca = pltpu.async_copy(atom_hbm.at[aidx.at[k]], abuf, sema)
        cp = pltpu.async_copy(pos_hbm.at[pidx.at[k]], pbuf, semp)
        ca.wait()
        cp.wait()

        def add_row(t, _):
            for j in range(H // LANES):
                sl = pl.ds(j * LANES, LANES)
                abuf[t, sl] = abuf[t, sl] + pbuf[t, sl]
            return 0

        lax.fori_loop(0, CHUNK, add_row, 0)
        pltpu.async_copy(abuf, out_hbm.at[oidx.at[k]], semo).wait()

    gidx[pl.ds(0, LANES)] = iota * 0 + b * (L + 1)
    for r in range(LANES):
        pltpu.sync_copy(gt_hbm.at[pl.ds(0, 1)], gtbuf.at[pl.ds(r, 1)])
    pltpu.async_copy(gtbuf, out_hbm.at[gidx], semo).wait()


def kernel(x, atom_table, pos_table, graph_token):
    positions = _positions(x)
    mesh = plsc.VectorSubcoreMesh(
        core_axis_name="c", subcore_axis_name="s", num_cores=2, num_subcores=16)
    f = pl.kernel(
        _sc_body,
        out_type=jax.ShapeDtypeStruct((B * (L + 1), H), jnp.float32),
        mesh=mesh,
        scratch_types=[
            pltpu.VMEM((NCHUNK, CHUNK), jnp.int32),
            pltpu.VMEM((NCHUNK, CHUNK), jnp.int32),
            pltpu.VMEM((NCHUNK, CHUNK), jnp.int32),
            pltpu.VMEM((LANES,), jnp.int32),
            pltpu.VMEM((CHUNK, H), jnp.float32),
            pltpu.VMEM((CHUNK, H), jnp.float32),
            pltpu.VMEM((LANES, H), jnp.float32),
            pltpu.SemaphoreType.DMA,
            pltpu.SemaphoreType.DMA,
            pltpu.SemaphoreType.DMA,
        ],
    )
    out = f(x, positions, atom_table, pos_table, graph_token)
    return out.reshape(B, L + 1, H)
